# baseline (speedup 1.0000x reference)
import functools

import jax
import jax.numpy as jnp
from jax import lax
from jax.experimental import pallas as pl
from jax.experimental.pallas import tpu as pltpu
from jax.experimental.pallas import tpu_sc as plsc

_N_PATTERNS = 11
_EMBED = 64
_BATCH = 16384

_info = plsc.get_sparse_core_info()
_NC, _NS, _L = _info.num_cores, _info.num_subcores, _info.num_lanes
_NW = _NC * _NS
_BPW = _BATCH // _NW

_mesh = plsc.VectorSubcoreMesh(core_axis_name="c", subcore_axis_name="s")


@functools.partial(
    pl.kernel,
    mesh=_mesh,
    compiler_params=pltpu.CompilerParams(
        use_tc_tiling_on_sc=False,
        needs_layout_passes=False,
    ),
    out_type=(
        jax.ShapeDtypeStruct((_BATCH, _EMBED), jnp.float32),
        jax.ShapeDtypeStruct((_BATCH,), jnp.float32),
    ),
    scratch_types=[
        pltpu.VMEM((_BPW,), jnp.int32),
        pltpu.VMEM((_BPW, _EMBED), jnp.float32),
        pltpu.VMEM((_BPW,), jnp.float32),
        pltpu.SemaphoreType.DMA,
        pltpu.SemaphoreType.DMA,
    ],
)
def _sc_lookup(ids_hbm, table_hbm, w_hbm, out_hbm, wout_hbm,
               idx_v, rows_v, wout_v, sem, wsem):
    wid = lax.axis_index("s") * _NC + lax.axis_index("c")
    base = wid * _BPW
    pltpu.sync_copy(ids_hbm.at[pl.ds(base, _BPW)], idx_v)
    gather = pltpu.async_copy(table_hbm.at[idx_v], rows_v, sem)
    wgather = pltpu.async_copy(w_hbm.at[idx_v], wout_v, wsem)
    wgather.wait()
    pltpu.sync_copy(wout_v, wout_hbm.at[pl.ds(base, _BPW)])
    gather.wait()
    pltpu.sync_copy(rows_v, out_hbm.at[pl.ds(base, _BPW)])


def kernel(pattern_ids, pattern_embeddings, pattern_weights):
    ids = pattern_ids.astype(jnp.int32)
    emb, w = _sc_lookup(ids, pattern_embeddings, pattern_weights)
    return emb, w


# speedup vs baseline: 1.5971x; 1.5971x over previous
import functools

import jax
import jax.numpy as jnp
from jax import lax
from jax.experimental import pallas as pl
from jax.experimental.pallas import tpu as pltpu
from jax.experimental.pallas import tpu_sc as plsc

_N_PATTERNS = 11
_EMBED = 64
_BATCH = 16384

_info = plsc.get_sparse_core_info()
_NC, _NS, _L = _info.num_cores, _info.num_subcores, _info.num_lanes
_NW = _NC * _NS
_BPW = _BATCH // _NW

_mesh = plsc.VectorSubcoreMesh(core_axis_name="c", subcore_axis_name="s")


@functools.partial(
    pl.kernel,
    mesh=_mesh,
    compiler_params=pltpu.CompilerParams(
        use_tc_tiling_on_sc=False,
        needs_layout_passes=False,
    ),
    out_type=(
        jax.ShapeDtypeStruct((_BATCH, _EMBED), jnp.float32),
        jax.ShapeDtypeStruct((_BATCH,), jnp.float32),
    ),
    scratch_types=[
        pltpu.VMEM((_BPW,), jnp.int32),
        pltpu.VMEM((_N_PATTERNS, _EMBED), jnp.float32),
        pltpu.VMEM((_N_PATTERNS,), jnp.float32),
        pltpu.VMEM((_BPW, _EMBED), jnp.float32),
        pltpu.VMEM((_BPW,), jnp.float32),
    ],
)
def _sc_lookup(ids_hbm, table_hbm, w_hbm, out_hbm, wout_hbm,
               idx_v, tab_v, wtab_v, rows_v, wout_v):
    wid = lax.axis_index("s") * _NC + lax.axis_index("c")
    base = wid * _BPW
    pltpu.sync_copy(ids_hbm.at[pl.ds(base, _BPW)], idx_v)
    pltpu.sync_copy(table_hbm, tab_v)
    pltpu.sync_copy(w_hbm, wtab_v)

    lane = jnp.arange(_L, dtype=jnp.int32)
    for g in range(_BPW // _L):
        id_vec = idx_v[pl.ds(g * _L, _L)]
        wout_v[pl.ds(g * _L, _L)] = plsc.load_gather(wtab_v, [id_vec])
        rvec = lane + (g * _L)
        for c in range(_EMBED):
            cvec = jnp.full((_L,), c, dtype=jnp.int32)
            val = plsc.load_gather(tab_v, [id_vec, cvec])
            plsc.store_scatter(rows_v, [rvec, cvec], val)

    pltpu.sync_copy(wout_v, wout_hbm.at[pl.ds(base, _BPW)])
    pltpu.sync_copy(rows_v, out_hbm.at[pl.ds(base, _BPW)])


def kernel(pattern_ids, pattern_embeddings, pattern_weights):
    ids = pattern_ids.astype(jnp.int32)
    emb, w = _sc_lookup(ids, pattern_embeddings, pattern_weights)
    return emb, w


# speedup vs baseline: 2.9832x; 1.8679x over previous
import functools

import jax
import jax.numpy as jnp
from jax import lax
from jax.experimental import pallas as pl
from jax.experimental.pallas import tpu as pltpu
from jax.experimental.pallas import tpu_sc as plsc

_N_PATTERNS = 11
_EMBED = 64
_BATCH = 16384

_info = plsc.get_sparse_core_info()
_NC, _NS, _L = _info.num_cores, _info.num_subcores, _info.num_lanes
_NW = _NC * _NS
_BPW = _BATCH // _NW

_mesh = plsc.VectorSubcoreMesh(core_axis_name="c", subcore_axis_name="s")


@functools.partial(
    pl.kernel,
    mesh=_mesh,
    compiler_params=pltpu.CompilerParams(
        use_tc_tiling_on_sc=False,
        needs_layout_passes=False,
    ),
    out_type=(
        jax.ShapeDtypeStruct((_BATCH, _EMBED), jnp.float32),
        jax.ShapeDtypeStruct((_BATCH,), jnp.float32),
    ),
    scratch_types=[
        pltpu.VMEM((_BPW,), jnp.int32),
        pltpu.VMEM((_N_PATTERNS, _EMBED), jnp.float32),
        pltpu.VMEM((_N_PATTERNS,), jnp.float32),
        pltpu.VMEM((_BPW, _EMBED), jnp.float32),
        pltpu.VMEM((_BPW,), jnp.float32),
    ],
)
def _sc_lookup(ids_hbm, table_hbm, w_hbm, out_hbm, wout_hbm,
               idx_v, tab_v, wtab_v, rows_v, wout_v):
    wid = lax.axis_index("s") * _NC + lax.axis_index("c")
    base = wid * _BPW
    pltpu.sync_copy(ids_hbm.at[pl.ds(base, _BPW)], idx_v)
    pltpu.sync_copy(table_hbm, tab_v)
    pltpu.sync_copy(w_hbm, wtab_v)

    lane = jnp.arange(_L, dtype=jnp.int32)

    @pl.loop(0, _BPW // _L)
    def _group(g):
        id_vec = idx_v[pl.ds(g * _L, _L)]
        wout_v[pl.ds(g * _L, _L)] = plsc.load_gather(wtab_v, [id_vec])
        rvec = lane + (g * _L)
        for c in range(_EMBED):
            cvec = (lane + c) & (_EMBED - 1)
            val = plsc.load_gather(tab_v, [id_vec, cvec])
            plsc.store_scatter(rows_v, [rvec, cvec], val)

    pltpu.sync_copy(wout_v, wout_hbm.at[pl.ds(base, _BPW)])
    pltpu.sync_copy(rows_v, out_hbm.at[pl.ds(base, _BPW)])


def kernel(pattern_ids, pattern_embeddings, pattern_weights):
    ids = pattern_ids.astype(jnp.int32)
    emb, w = _sc_lookup(ids, pattern_embeddings, pattern_weights)
    return emb, w


# speedup vs baseline: 3.4480x; 1.1558x over previous
import functools

import jax
import jax.numpy as jnp
from jax import lax
from jax.experimental import pallas as pl
from jax.experimental.pallas import tpu as pltpu
from jax.experimental.pallas import tpu_sc as plsc

_N_PATTERNS = 11
_EMBED = 64
_BATCH = 16384

_info = plsc.get_sparse_core_info()
_NC, _NS, _L = _info.num_cores, _info.num_subcores, _info.num_lanes
_NW = _NC * _NS
_BPW = _BATCH // _NW

_mesh = plsc.VectorSubcoreMesh(core_axis_name="c", subcore_axis_name="s")


@functools.partial(
    pl.kernel,
    mesh=_mesh,
    compiler_params=pltpu.CompilerParams(
        use_tc_tiling_on_sc=True,
        needs_layout_passes=False,
    ),
    out_type=(
        jax.ShapeDtypeStruct((_BATCH, _EMBED), jnp.float32),
        jax.ShapeDtypeStruct((_BATCH,), jnp.float32),
    ),
    scratch_types=[
        pltpu.VMEM((_BPW,), jnp.int32),
        pltpu.VMEM((_N_PATTERNS, _EMBED), jnp.float32),
        pltpu.VMEM((_N_PATTERNS,), jnp.float32),
        pltpu.VMEM((_BPW, _EMBED), jnp.float32),
        pltpu.VMEM((_BPW,), jnp.float32),
    ],
)
def _sc_lookup(ids_hbm, table_hbm, w_hbm, out_hbm, wout_hbm,
               idx_v, tab_v, wtab_v, rows_v, wout_v):
    wid = lax.axis_index("s") * _NC + lax.axis_index("c")
    base = wid * _BPW
    pltpu.sync_copy(ids_hbm.at[pl.ds(base, _BPW)], idx_v)
    pltpu.sync_copy(table_hbm, tab_v)
    pltpu.sync_copy(w_hbm, wtab_v)

    lane = jnp.arange(_L, dtype=jnp.int32)

    @pl.loop(0, _BPW // _L)
    def _group(g):
        id_vec = idx_v[pl.ds(g * _L, _L)]
        wout_v[pl.ds(g * _L, _L)] = plsc.load_gather(wtab_v, [id_vec])
        rvec = lane + (g * _L)
        for c in range(_EMBED):
            cvec = (lane + c) & (_EMBED - 1)
            val = plsc.load_gather(tab_v, [id_vec, cvec])
            plsc.store_scatter(rows_v, [rvec, cvec], val)

    pltpu.sync_copy(wout_v, wout_hbm.at[pl.ds(base, _BPW)])
    pltpu.sync_copy(rows_v, out_hbm.at[pl.ds(base, _BPW)])


def kernel(pattern_ids, pattern_embeddings, pattern_weights):
    ids = pattern_ids.astype(jnp.int32)
    emb, w = _sc_lookup(ids, pattern_embeddings, pattern_weights)
    return emb, w


# speedup vs baseline: 4.1013x; 1.1895x over previous
import functools

import jax
import jax.numpy as jnp
from jax import lax
from jax.experimental import pallas as pl
from jax.experimental.pallas import tpu as pltpu
from jax.experimental.pallas import tpu_sc as plsc

_N_PATTERNS = 11
_EMBED = 64
_BATCH = 16384

_info = plsc.get_sparse_core_info()
_NC, _NS, _L = _info.num_cores, _info.num_subcores, _info.num_lanes
_NW = _NC * _NS
_BPW = _BATCH // _NW

_mesh = plsc.VectorSubcoreMesh(core_axis_name="c", subcore_axis_name="s")


@functools.partial(
    pl.kernel,
    mesh=_mesh,
    compiler_params=pltpu.CompilerParams(
        use_tc_tiling_on_sc=True,
        needs_layout_passes=False,
    ),
    out_type=(
        jax.ShapeDtypeStruct((_EMBED, _BATCH), jnp.float32),
        jax.ShapeDtypeStruct((_BATCH,), jnp.float32),
    ),
    scratch_types=[
        pltpu.VMEM((_BPW,), jnp.int32),
        pltpu.VMEM((_N_PATTERNS, _EMBED), jnp.float32),
        pltpu.VMEM((_N_PATTERNS,), jnp.float32),
        pltpu.VMEM((_EMBED, _BPW), jnp.float32),
        pltpu.VMEM((_BPW,), jnp.float32),
    ],
)
def _sc_lookup(ids_hbm, table_hbm, w_hbm, out_hbm, wout_hbm,
               idx_v, tab_v, wtab_v, rows_v, wout_v):
    wid = lax.axis_index("s") * _NC + lax.axis_index("c")
    base = wid * _BPW
    pltpu.sync_copy(ids_hbm.at[pl.ds(base, _BPW)], idx_v)
    pltpu.sync_copy(table_hbm, tab_v)
    pltpu.sync_copy(w_hbm, wtab_v)

    lane = jnp.arange(_L, dtype=jnp.int32)

    @pl.loop(0, _BPW // _L)
    def _group(g):
        id_vec = idx_v[pl.ds(g * _L, _L)]
        wout_v[pl.ds(g * _L, _L)] = plsc.load_gather(wtab_v, [id_vec])
        rvec = lane + (g * _L)
        for c in range(_EMBED):
            cvec = (lane + c) & (_EMBED - 1)
            val = plsc.load_gather(tab_v, [id_vec, cvec])
            plsc.store_scatter(rows_v, [cvec, rvec], val)

    pltpu.sync_copy(wout_v, wout_hbm.at[pl.ds(base, _BPW)])
    pltpu.sync_copy(rows_v, out_hbm.at[:, pl.ds(base, _BPW)])


def kernel(pattern_ids, pattern_embeddings, pattern_weights):
    ids = pattern_ids.astype(jnp.int32)
    emb_t, w = _sc_lookup(ids, pattern_embeddings, pattern_weights)
    return emb_t.T, w


# speedup vs baseline: 4.3808x; 1.0682x over previous
import functools

import jax
import jax.numpy as jnp
from jax import lax
from jax.experimental import pallas as pl
from jax.experimental.pallas import tpu as pltpu
from jax.experimental.pallas import tpu_sc as plsc

_N_PATTERNS = 11
_EMBED = 64
_BATCH = 16384

_info = plsc.get_sparse_core_info()
_NC, _NS, _L = _info.num_cores, _info.num_subcores, _info.num_lanes
_NW = _NC * _NS
_BPW = _BATCH // _NW

_mesh = plsc.VectorSubcoreMesh(core_axis_name="c", subcore_axis_name="s")


@functools.partial(
    pl.kernel,
    mesh=_mesh,
    compiler_params=pltpu.CompilerParams(
        use_tc_tiling_on_sc=True,
        needs_layout_passes=False,
    ),
    out_type=(
        jax.ShapeDtypeStruct((_EMBED, _BATCH), jnp.float32),
        jax.ShapeDtypeStruct((_BATCH,), jnp.float32),
    ),
    scratch_types=[
        pltpu.VMEM((_BPW,), jnp.int32),
        pltpu.VMEM((_N_PATTERNS, _EMBED), jnp.float32),
        pltpu.VMEM((_N_PATTERNS,), jnp.float32),
        pltpu.VMEM((_EMBED, _BPW), jnp.float32),
        pltpu.VMEM((_BPW,), jnp.float32),
    ],
)
def _sc_lookup(ids_hbm, table_hbm, w_hbm, out_hbm, wout_hbm,
               idx_v, tab_v, wtab_v, rows_v, wout_v):
    wid = lax.axis_index("s") * _NC + lax.axis_index("c")
    base = wid * _BPW
    pltpu.sync_copy(ids_hbm.at[pl.ds(base, _BPW)], idx_v)
    pltpu.sync_copy(table_hbm, tab_v)
    pltpu.sync_copy(w_hbm, wtab_v)

    lane = jnp.arange(_L, dtype=jnp.int32)

    @plsc.parallel_loop(0, _BPW // _L, unroll=2)
    def _group(g):
        id_vec = idx_v[pl.ds(g * _L, _L)]
        wout_v[pl.ds(g * _L, _L)] = plsc.load_gather(wtab_v, [id_vec])
        rvec = lane + (g * _L)
        for c in range(_EMBED):
            cvec = (lane + c) & (_EMBED - 1)
            val = plsc.load_gather(tab_v, [id_vec, cvec])
            plsc.store_scatter(rows_v, [cvec, rvec], val)

    pltpu.sync_copy(wout_v, wout_hbm.at[pl.ds(base, _BPW)])
    pltpu.sync_copy(rows_v, out_hbm.at[:, pl.ds(base, _BPW)])


def kernel(pattern_ids, pattern_embeddings, pattern_weights):
    ids = pattern_ids.astype(jnp.int32)
    emb_t, w = _sc_lookup(ids, pattern_embeddings, pattern_weights)
    return emb_t.T, w


# speedup vs baseline: 4.3843x; 1.0008x over previous
import functools

import jax
import jax.numpy as jnp
from jax import lax
from jax.experimental import pallas as pl
from jax.experimental.pallas import tpu as pltpu
from jax.experimental.pallas import tpu_sc as plsc

_N_PATTERNS = 11
_EMBED = 64
_BATCH = 16384

_info = plsc.get_sparse_core_info()
_NC, _NS, _L = _info.num_cores, _info.num_subcores, _info.num_lanes
_NW = _NC * _NS
_BPW = _BATCH // _NW

_mesh = plsc.VectorSubcoreMesh(core_axis_name="c", subcore_axis_name="s")


@functools.partial(
    pl.kernel,
    mesh=_mesh,
    compiler_params=pltpu.CompilerParams(
        use_tc_tiling_on_sc=True,
        needs_layout_passes=False,
    ),
    out_type=(
        jax.ShapeDtypeStruct((_EMBED, _BATCH), jnp.float32),
        jax.ShapeDtypeStruct((_BATCH,), jnp.float32),
    ),
    scratch_types=[
        pltpu.VMEM((_BPW,), jnp.int32),
        pltpu.VMEM((_N_PATTERNS, _EMBED), jnp.float32),
        pltpu.VMEM((_N_PATTERNS,), jnp.float32),
        pltpu.VMEM((_EMBED, _BPW), jnp.float32),
        pltpu.VMEM((_BPW,), jnp.float32),
    ],
)
def _sc_lookup(ids_hbm, table_hbm, w_hbm, out_hbm, wout_hbm,
               idx_v, tab_v, wtab_v, rows_v, wout_v):
    wid = lax.axis_index("s") * _NC + lax.axis_index("c")
    base = wid * _BPW
    pltpu.sync_copy(ids_hbm.at[pl.ds(base, _BPW)], idx_v)
    pltpu.sync_copy(table_hbm, tab_v)
    pltpu.sync_copy(w_hbm, wtab_v)

    lane = jnp.arange(_L, dtype=jnp.int32)

    @plsc.parallel_loop(0, _BPW // _L, unroll=1)
    def _group(g):
        id_vec = idx_v[pl.ds(g * _L, _L)]
        wout_v[pl.ds(g * _L, _L)] = plsc.load_gather(wtab_v, [id_vec])
        rvec = lane + (g * _L)
        for c in range(_EMBED):
            cvec = (lane + c) & (_EMBED - 1)
            val = plsc.load_gather(tab_v, [id_vec, cvec])
            plsc.store_scatter(rows_v, [cvec, rvec], val)

    pltpu.sync_copy(wout_v, wout_hbm.at[pl.ds(base, _BPW)])
    pltpu.sync_copy(rows_v, out_hbm.at[:, pl.ds(base, _BPW)])


def kernel(pattern_ids, pattern_embeddings, pattern_weights):
    ids = pattern_ids.astype(jnp.int32)
    emb_t, w = _sc_lookup(ids, pattern_embeddings, pattern_weights)
    return emb_t.T, w


# speedup vs baseline: 5.0497x; 1.1518x over previous
import functools

import jax
import jax.numpy as jnp
from jax import lax
from jax.experimental import pallas as pl
from jax.experimental.pallas import tpu as pltpu
from jax.experimental.pallas import tpu_sc as plsc

_N_PATTERNS = 11
_EMBED = 64
_BATCH = 16384

_info = plsc.get_sparse_core_info()
_NC, _NS, _L = _info.num_cores, _info.num_subcores, _info.num_lanes
_NW = _NC * _NS
_BPW = _BATCH // _NW

_mesh = plsc.VectorSubcoreMesh(core_axis_name="c", subcore_axis_name="s")


@functools.partial(
    pl.kernel,
    mesh=_mesh,
    compiler_params=pltpu.CompilerParams(
        use_tc_tiling_on_sc=True,
        needs_layout_passes=False,
    ),
    out_type=(
        jax.ShapeDtypeStruct((_EMBED, _BATCH), jnp.float32),
        jax.ShapeDtypeStruct((_BATCH,), jnp.float32),
    ),
    scratch_types=[
        pltpu.VMEM((_BPW,), jnp.int32),
        pltpu.VMEM((_N_PATTERNS, _EMBED), jnp.float32),
        pltpu.VMEM((_N_PATTERNS,), jnp.float32),
        pltpu.VMEM((_EMBED, _BPW), jnp.float32),
        pltpu.VMEM((_BPW,), jnp.float32),
    ],
)
def _sc_lookup(ids_hbm, table_hbm, w_hbm, out_hbm, wout_hbm,
               idx_v, tab_v, wtab_v, rows_v, wout_v):
    wid = lax.axis_index("s") * _NC + lax.axis_index("c")
    base = wid * _BPW
    pltpu.sync_copy(ids_hbm.at[pl.ds(base, _BPW)], idx_v)
    pltpu.sync_copy(table_hbm, tab_v)
    pltpu.sync_copy(w_hbm, wtab_v)

    lane = jnp.arange(_L, dtype=jnp.int32)

    @plsc.parallel_loop(0, _BPW // _L, unroll=1)
    def _wgroup(g):
        id_vec = idx_v[pl.ds(g * _L, _L)]
        wout_v[pl.ds(g * _L, _L)] = plsc.load_gather(wtab_v, [id_vec])

    _CQ = 16

    @plsc.parallel_loop(0, (_BPW // _L) * (_EMBED // _CQ), unroll=1)
    def _group(t):
        g = t // (_EMBED // _CQ)
        q = t % (_EMBED // _CQ)
        id_vec = idx_v[pl.ds(g * _L, _L)]
        rvec = lane + (g * _L)
        cbase = q * _CQ
        for c in range(_CQ):
            cvec = (lane + c + cbase) & (_EMBED - 1)
            val = plsc.load_gather(tab_v, [id_vec, cvec])
            plsc.store_scatter(rows_v, [cvec, rvec], val)

    pltpu.sync_copy(wout_v, wout_hbm.at[pl.ds(base, _BPW)])
    pltpu.sync_copy(rows_v, out_hbm.at[:, pl.ds(base, _BPW)])


def kernel(pattern_ids, pattern_embeddings, pattern_weights):
    ids = pattern_ids.astype(jnp.int32)
    emb_t, w = _sc_lookup(ids, pattern_embeddings, pattern_weights)
    return emb_t.T, w


# speedup vs baseline: 5.0565x; 1.0013x over previous
import functools

import jax
import jax.numpy as jnp
from jax import lax
from jax.experimental import pallas as pl
from jax.experimental.pallas import tpu as pltpu
from jax.experimental.pallas import tpu_sc as plsc

_N_PATTERNS = 11
_EMBED = 64
_BATCH = 16384

_info = plsc.get_sparse_core_info()
_NC, _NS, _L = _info.num_cores, _info.num_subcores, _info.num_lanes
_NW = _NC * _NS
_BPW = _BATCH // _NW

_mesh = plsc.VectorSubcoreMesh(core_axis_name="c", subcore_axis_name="s")


@functools.partial(
    pl.kernel,
    mesh=_mesh,
    compiler_params=pltpu.CompilerParams(
        use_tc_tiling_on_sc=True,
        needs_layout_passes=False,
    ),
    out_type=(
        jax.ShapeDtypeStruct((_EMBED, _BATCH), jnp.float32),
        jax.ShapeDtypeStruct((_BATCH,), jnp.float32),
    ),
    scratch_types=[
        pltpu.VMEM((_BPW,), jnp.int32),
        pltpu.VMEM((_N_PATTERNS, _EMBED), jnp.float32),
        pltpu.VMEM((_N_PATTERNS,), jnp.float32),
        pltpu.VMEM((_EMBED, _BPW), jnp.float32),
        pltpu.VMEM((_BPW,), jnp.float32),
    ],
)
def _sc_lookup(ids_hbm, table_hbm, w_hbm, out_hbm, wout_hbm,
               idx_v, tab_v, wtab_v, rows_v, wout_v):
    wid = lax.axis_index("s") * _NC + lax.axis_index("c")
    base = wid * _BPW
    pltpu.sync_copy(ids_hbm.at[pl.ds(base, _BPW)], idx_v)
    pltpu.sync_copy(table_hbm, tab_v)
    pltpu.sync_copy(w_hbm, wtab_v)

    lane = jnp.arange(_L, dtype=jnp.int32)

    @plsc.parallel_loop(0, _BPW // _L, unroll=1)
    def _wgroup(g):
        id_vec = idx_v[pl.ds(g * _L, _L)]
        wout_v[pl.ds(g * _L, _L)] = plsc.load_gather(wtab_v, [id_vec])

    _CQ = 8

    @plsc.parallel_loop(0, (_BPW // _L) * (_EMBED // _CQ), unroll=1)
    def _group(t):
        g = t // (_EMBED // _CQ)
        q = t % (_EMBED // _CQ)
        id_vec = idx_v[pl.ds(g * _L, _L)]
        rvec = lane + (g * _L)
        cbase = q * _CQ
        for c in range(_CQ):
            cvec = (lane + c + cbase) & (_EMBED - 1)
            val = plsc.load_gather(tab_v, [id_vec, cvec])
            plsc.store_scatter(rows_v, [cvec, rvec], val)

    pltpu.sync_copy(wout_v, wout_hbm.at[pl.ds(base, _BPW)])
    pltpu.sync_copy(rows_v, out_hbm.at[:, pl.ds(base, _BPW)])


def kernel(pattern_ids, pattern_embeddings, pattern_weights):
    ids = pattern_ids.astype(jnp.int32)
    emb_t, w = _sc_lookup(ids, pattern_embeddings, pattern_weights)
    return emb_t.T, w
